# Initial kernel scaffold; baseline (speedup 1.0000x reference)
import functools

import jax
import jax.numpy as jnp
from jax import lax
from jax.experimental import pallas as pl
from jax.experimental.pallas import tpu as pltpu
from jax.experimental.pallas import tpu_sc as plsc

N = 10000
E = 320000
DF = 128
DE = 16
DT = 16
DO = 128

NC = 2
NS = 16
ROWS_PER_TILE = N // NS
EDGES_PER_CORE = E // NC
EDGES_PER_TILE = EDGES_PER_CORE // NS
CHUNK = 80
NCHUNKS = EDGES_PER_TILE // CHUNK


def _sc_aggregate(row, col, x, edge_attr, edge_t, z128, z16):
  mesh = plsc.VectorSubcoreMesh(core_axis_name="c", subcore_axis_name="s")

  @functools.partial(
      pl.kernel,
      out_type=(
          jax.ShapeDtypeStruct((NC, N, DF), jnp.float32),
          jax.ShapeDtypeStruct((NC, N, DE), jnp.float32),
          jax.ShapeDtypeStruct((NC, N, DT), jnp.float32),
      ),
      mesh=mesh,
      scratch_types=[
          pltpu.VMEM_SHARED((N, DF), jnp.float32),
          pltpu.VMEM_SHARED((N, DE), jnp.float32),
          pltpu.VMEM_SHARED((N, DT), jnp.float32),
          pltpu.VMEM((CHUNK,), jnp.int32),
          pltpu.VMEM((CHUNK,), jnp.int32),
          pltpu.VMEM((CHUNK, DF), jnp.float32),
          pltpu.VMEM((CHUNK, DE), jnp.float32),
          pltpu.VMEM((CHUNK, DT), jnp.float32),
          pltpu.SemaphoreType.DMA,
      ],
  )
  def agg_kernel(row_h, col_h, x_h, ea_h, et_h, z128_h, z16_h,
                 px_h, pa_h, pt_h,
                 aggx, agga, aggt, row_v, col_v, xr_v, ea_v, et_v, sem):
    c = lax.axis_index("c")
    s = lax.axis_index("s")
    rbase = s * ROWS_PER_TILE

    pltpu.sync_copy(z128_h, aggx.at[pl.ds(rbase, ROWS_PER_TILE)])
    pltpu.sync_copy(z16_h, agga.at[pl.ds(rbase, ROWS_PER_TILE)])
    pltpu.sync_copy(z16_h, aggt.at[pl.ds(rbase, ROWS_PER_TILE)])
    plsc.subcore_barrier()

    ebase = c * EDGES_PER_CORE + s * EDGES_PER_TILE

    @pl.loop(0, NCHUNKS)
    def _(j):
      off = ebase + j * CHUNK
      pltpu.sync_copy(row_h.at[pl.ds(off, CHUNK)], row_v)
      pltpu.sync_copy(col_h.at[pl.ds(off, CHUNK)], col_v)
      pltpu.async_copy(x_h.at[col_v], xr_v, sem).wait()
      pltpu.sync_copy(ea_h.at[pl.ds(off, CHUNK)], ea_v)
      pltpu.sync_copy(et_h.at[pl.ds(off, CHUNK)], et_v)
      pltpu.sync_copy(xr_v, aggx.at[row_v], add=True)
      pltpu.sync_copy(ea_v, agga.at[row_v], add=True)
      pltpu.sync_copy(et_v, aggt.at[row_v], add=True)

    plsc.subcore_barrier()
    pltpu.sync_copy(aggx.at[pl.ds(rbase, ROWS_PER_TILE)],
                    px_h.at[c, pl.ds(rbase, ROWS_PER_TILE)])
    pltpu.sync_copy(agga.at[pl.ds(rbase, ROWS_PER_TILE)],
                    pa_h.at[c, pl.ds(rbase, ROWS_PER_TILE)])
    pltpu.sync_copy(aggt.at[pl.ds(rbase, ROWS_PER_TILE)],
                    pt_h.at[c, pl.ds(rbase, ROWS_PER_TILE)])

  return agg_kernel(row, col, x, edge_attr, edge_t, z128, z16)


BLK = 1000


def _tc_combine(px, pa, pt, x, wmx, wma, wmt, wr, bm, br):
  def body(px_r, pa_r, pt_r, x_r, wmx_r, wma_r, wmt_r, wr_r, bm_r, br_r, o_r):
    aggx = px_r[0] + px_r[1]
    agga = pa_r[0] + pa_r[1]
    aggt = pt_r[0] + pt_r[1]
    acc = jnp.dot(aggx, wmx_r[...], preferred_element_type=jnp.float32)
    acc = acc + jnp.dot(agga, wma_r[...], preferred_element_type=jnp.float32)
    acc = acc + jnp.dot(aggt, wmt_r[...], preferred_element_type=jnp.float32)
    acc = 0.5 * (acc + bm_r[...])
    acc = acc + jnp.dot(x_r[...], wr_r[...], preferred_element_type=jnp.float32)
    o_r[...] = acc + br_r[...]

  return pl.pallas_call(
      body,
      grid=(N // BLK,),
      in_specs=[
          pl.BlockSpec((NC, BLK, DF), lambda i: (0, i, 0)),
          pl.BlockSpec((NC, BLK, DE), lambda i: (0, i, 0)),
          pl.BlockSpec((NC, BLK, DT), lambda i: (0, i, 0)),
          pl.BlockSpec((BLK, DF), lambda i: (i, 0)),
          pl.BlockSpec((DF, DO), lambda i: (0, 0)),
          pl.BlockSpec((DE, DO), lambda i: (0, 0)),
          pl.BlockSpec((DT, DO), lambda i: (0, 0)),
          pl.BlockSpec((DF, DO), lambda i: (0, 0)),
          pl.BlockSpec((1, DO), lambda i: (0, 0)),
          pl.BlockSpec((1, DO), lambda i: (0, 0)),
      ],
      out_specs=pl.BlockSpec((BLK, DO), lambda i: (i, 0)),
      out_shape=jax.ShapeDtypeStruct((N, DO), jnp.float32),
  )(px, pa, pt, x, wmx, wma, wmt, wr, bm, br)


def kernel(x, edge_index, edge_attr, edge_t, W_m, b_m, W_r, b_r):
  ei = edge_index.astype(jnp.int32)
  row = ei[0]
  col = ei[1]
  z128 = jnp.zeros((ROWS_PER_TILE, DF), jnp.float32)
  z16 = jnp.zeros((ROWS_PER_TILE, DE), jnp.float32)
  px, pa, pt = _sc_aggregate(row, col, x, edge_attr, edge_t, z128, z16)
  wmx = W_m[:DF]
  wma = W_m[DF:DF + DE]
  wmt = W_m[DF + DE:]
  bm = b_m.reshape(1, DO)
  br = b_r.reshape(1, DO)
  return _tc_combine(px, pa, pt, x, wmx, wma, wmt, W_r, bm, br)


# speedup vs baseline: 3.8560x; 3.8560x over previous
import functools

import jax
import jax.numpy as jnp
from jax import lax
from jax.experimental import pallas as pl
from jax.experimental.pallas import tpu as pltpu
from jax.experimental.pallas import tpu_sc as plsc

N = 10000
E = 320000
DF = 128
DE = 16
DT = 16
DO = 128

NC = 2
NS = 16
NPAD = 10240
ROWS_PER_TILE = NPAD // NS
EDGES_PER_CORE = E // NC
EDGES_PER_TILE = EDGES_PER_CORE // NS
CHUNK = 80
NCHUNKS = EDGES_PER_TILE // CHUNK


def _sc_aggregate(row, col, x, edge_at):
  mesh = plsc.VectorSubcoreMesh(core_axis_name="c", subcore_axis_name="s")

  @functools.partial(
      pl.kernel,
      out_type=(
          jax.ShapeDtypeStruct((NC, NPAD, DF), jnp.float32),
          jax.ShapeDtypeStruct((NC, NPAD, DE + DT), jnp.float32),
      ),
      mesh=mesh,
      compiler_params=pltpu.CompilerParams(use_tc_tiling_on_sc=False),
      scratch_types=[
          pltpu.VMEM_SHARED((NPAD, DF), jnp.float32),
          pltpu.VMEM_SHARED((NPAD, DE + DT), jnp.float32),
          pltpu.VMEM((CHUNK,), jnp.int32),
          pltpu.VMEM((CHUNK,), jnp.int32),
          pltpu.VMEM((CHUNK, DF), jnp.float32),
          pltpu.VMEM((CHUNK, DE + DT), jnp.float32),
          pltpu.SemaphoreType.DMA,
      ],
  )
  def agg_kernel(row_h, col_h, x_h, eat_h,
                 px_h, pat_h,
                 aggx, aggat, row_v, col_v, xr_v, eat_v, sem):
    c = lax.axis_index("c")
    s = lax.axis_index("s")
    rbase = s * ROWS_PER_TILE

    zeros16 = jnp.zeros((16,), jnp.float32)

    @pl.loop(0, CHUNK)
    def _(i):
      @pl.loop(0, (DE + DT) // 16)
      def _(k):
        eat_v[i, pl.ds(k * 16, 16)] = zeros16

      @pl.loop(0, DF // 16)
      def _(k):
        xr_v[i, pl.ds(k * 16, 16)] = zeros16

    @pl.loop(0, ROWS_PER_TILE // CHUNK)
    def _(k):
      dst = rbase + k * CHUNK
      pltpu.sync_copy(xr_v, aggx.at[pl.ds(dst, CHUNK)])
      pltpu.sync_copy(eat_v, aggat.at[pl.ds(dst, CHUNK)])

    plsc.subcore_barrier()

    ebase = c * EDGES_PER_CORE + s * EDGES_PER_TILE

    @pl.loop(0, NCHUNKS)
    def _(j):
      off = ebase + j * CHUNK
      pltpu.sync_copy(row_h.at[pl.ds(off, CHUNK)], row_v)
      pltpu.sync_copy(col_h.at[pl.ds(off, CHUNK)], col_v)
      pltpu.async_copy(x_h.at[col_v], xr_v, sem).wait()
      pltpu.sync_copy(eat_h.at[pl.ds(off, CHUNK)], eat_v)
      pltpu.sync_copy(xr_v, aggx.at[row_v], add=True)
      pltpu.sync_copy(eat_v, aggat.at[row_v], add=True)

    plsc.subcore_barrier()
    pltpu.sync_copy(aggx.at[pl.ds(rbase, ROWS_PER_TILE)],
                    px_h.at[c, pl.ds(rbase, ROWS_PER_TILE)])
    pltpu.sync_copy(aggat.at[pl.ds(rbase, ROWS_PER_TILE)],
                    pat_h.at[c, pl.ds(rbase, ROWS_PER_TILE)])

  return agg_kernel(row, col, x, edge_at)


BLK = 1000


def _tc_combine(px, pat, x, wmx, wmat, wr, bm, br):
  def body(px_r, pat_r, x_r, wmx_r, wmat_r, wr_r, bm_r, br_r, o_r):
    aggx = px_r[0] + px_r[1]
    aggat = pat_r[0] + pat_r[1]
    acc = jnp.dot(aggx, wmx_r[...], preferred_element_type=jnp.float32)
    acc = acc + jnp.dot(aggat, wmat_r[...], preferred_element_type=jnp.float32)
    acc = 0.5 * (acc + bm_r[...])
    acc = acc + jnp.dot(x_r[...], wr_r[...], preferred_element_type=jnp.float32)
    o_r[...] = acc + br_r[...]

  return pl.pallas_call(
      body,
      grid=(N // BLK,),
      in_specs=[
          pl.BlockSpec((NC, BLK, DF), lambda i: (0, i, 0)),
          pl.BlockSpec((NC, BLK, DE + DT), lambda i: (0, i, 0)),
          pl.BlockSpec((BLK, DF), lambda i: (i, 0)),
          pl.BlockSpec((DF, DO), lambda i: (0, 0)),
          pl.BlockSpec((DE + DT, DO), lambda i: (0, 0)),
          pl.BlockSpec((DF, DO), lambda i: (0, 0)),
          pl.BlockSpec((1, DO), lambda i: (0, 0)),
          pl.BlockSpec((1, DO), lambda i: (0, 0)),
      ],
      out_specs=pl.BlockSpec((BLK, DO), lambda i: (i, 0)),
      out_shape=jax.ShapeDtypeStruct((N, DO), jnp.float32),
  )(px, pat, x, wmx, wmat, wr, bm, br)


def kernel(x, edge_index, edge_attr, edge_t, W_m, b_m, W_r, b_r):
  ei = edge_index.astype(jnp.int32)
  row = ei[0]
  col = ei[1]
  edge_at = jnp.concatenate([edge_attr, edge_t], axis=1)
  px, pat = _sc_aggregate(row, col, x, edge_at)
  px = px[:, :N]
  pat = pat[:, :N]
  wmx = W_m[:DF]
  wmat = W_m[DF:]
  bm = b_m.reshape(1, DO)
  br = b_r.reshape(1, DO)
  return _tc_combine(px, pat, x, wmx, wmat, W_r, bm, br)


# speedup vs baseline: 5.7315x; 1.4864x over previous
import functools

import jax
import jax.numpy as jnp
from jax import lax
from jax.experimental import pallas as pl
from jax.experimental.pallas import tpu as pltpu
from jax.experimental.pallas import tpu_sc as plsc

N = 10000
E = 320000
DF = 128
DE = 16
DT = 16
DO = 128

NC = 2
NS = 16
NPAD = 10240
ROWS_PER_TILE = NPAD // NS
EDGES_PER_CORE = E // NC
EDGES_PER_TILE = EDGES_PER_CORE // NS
CHUNK = 80
NCHUNKS = EDGES_PER_TILE // CHUNK


def _sc_aggregate(row, col, x, edge_at):
  mesh = plsc.VectorSubcoreMesh(core_axis_name="c", subcore_axis_name="s")

  @functools.partial(
      pl.kernel,
      out_type=(
          jax.ShapeDtypeStruct((NC, NPAD, DF), jnp.float32),
          jax.ShapeDtypeStruct((NC, NPAD, DE + DT), jnp.float32),
      ),
      mesh=mesh,
      compiler_params=pltpu.CompilerParams(use_tc_tiling_on_sc=False),
      scratch_types=[
          pltpu.VMEM_SHARED((NPAD, DF), jnp.float32),
          pltpu.VMEM_SHARED((NPAD, DE + DT), jnp.float32),
          pltpu.VMEM((CHUNK,), jnp.int32),
          pltpu.VMEM((CHUNK,), jnp.int32),
          pltpu.VMEM((CHUNK,), jnp.int32),
          pltpu.VMEM((CHUNK,), jnp.int32),
          pltpu.VMEM((CHUNK, DF), jnp.float32),
          pltpu.VMEM((CHUNK, DF), jnp.float32),
          pltpu.VMEM((CHUNK, DE + DT), jnp.float32),
          pltpu.VMEM((CHUNK, DE + DT), jnp.float32),
          pltpu.SemaphoreType.DMA,
          pltpu.SemaphoreType.DMA,
          pltpu.SemaphoreType.DMA,
          pltpu.SemaphoreType.DMA,
          pltpu.SemaphoreType.DMA,
          pltpu.SemaphoreType.DMA,
      ],
  )
  def agg_kernel(row_h, col_h, x_h, eat_h,
                 px_h, pat_h,
                 aggx, aggat, row_v0, row_v1, col_v0, col_v1,
                 xr_v0, xr_v1, eat_v0, eat_v1,
                 si0, si1, sg0, sg1, ss0, ss1):
    c = lax.axis_index("c")
    s = lax.axis_index("s")
    rbase = s * ROWS_PER_TILE

    rows = (row_v0, row_v1)
    cols = (col_v0, col_v1)
    xrs = (xr_v0, xr_v1)
    eats = (eat_v0, eat_v1)
    sis = (si0, si1)
    sgs = (sg0, sg1)
    sss = (ss0, ss1)

    zeros16 = jnp.zeros((16,), jnp.float32)

    @pl.loop(0, CHUNK)
    def _(i):
      @pl.loop(0, (DE + DT) // 16)
      def _(k):
        eat_v0[i, pl.ds(k * 16, 16)] = zeros16

      @pl.loop(0, DF // 16)
      def _(k):
        xr_v0[i, pl.ds(k * 16, 16)] = zeros16

    @pl.loop(0, ROWS_PER_TILE // CHUNK)
    def _(k):
      dst = rbase + k * CHUNK
      pltpu.sync_copy(xr_v0, aggx.at[pl.ds(dst, CHUNK)])
      pltpu.sync_copy(eat_v0, aggat.at[pl.ds(dst, CHUNK)])

    plsc.subcore_barrier()

    ebase = c * EDGES_PER_CORE + s * EDGES_PER_TILE

    def idx_start(j, b):
      off = ebase + j * CHUNK
      pltpu.async_copy(row_h.at[pl.ds(off, CHUNK)], rows[b], sis[b])
      pltpu.async_copy(col_h.at[pl.ds(off, CHUNK)], cols[b], sis[b])
      pltpu.async_copy(eat_h.at[pl.ds(off, CHUNK)], eats[b], sis[b])

    def idx_wait(b):
      pltpu.make_async_copy(row_h.at[pl.ds(0, CHUNK)], rows[b], sis[b]).wait()
      pltpu.make_async_copy(col_h.at[pl.ds(0, CHUNK)], cols[b], sis[b]).wait()
      pltpu.make_async_copy(eat_h.at[pl.ds(0, CHUNK)], eats[b], sis[b]).wait()

    def gather_start(b):
      pltpu.async_copy(x_h.at[cols[b]], xrs[b], sgs[b])

    def gather_wait(b):
      pltpu.make_async_copy(x_h.at[cols[b]], xrs[b], sgs[b]).wait()

    def scat_start(b):
      pltpu.async_copy(xrs[b], aggx.at[rows[b]], sss[b], add=True)
      pltpu.async_copy(eats[b], aggat.at[rows[b]], sss[b], add=True)

    def scat_wait(b):
      pltpu.make_async_copy(xrs[b], aggx.at[rows[b]], sss[b]).wait()
      pltpu.make_async_copy(eats[b], aggat.at[rows[b]], sss[b]).wait()

    idx_start(0, 0)
    idx_start(1, 1)

    @pl.loop(0, (NCHUNKS - 1) // 2)
    def _(i):
      a = 2 * i
      idx_wait(0)
      gather_start(0)
      idx_wait(1)
      gather_start(1)
      gather_wait(0)
      scat_start(0)
      gather_wait(1)
      scat_start(1)
      scat_wait(0)
      idx_start(a + 2, 0)
      scat_wait(1)
      idx_start(jnp.minimum(a + 3, NCHUNKS - 1), 1)

    idx_wait(0)
    gather_start(0)
    gather_wait(0)
    scat_start(0)
    scat_wait(0)
    idx_wait(1)

    plsc.subcore_barrier()
    pltpu.sync_copy(aggx.at[pl.ds(rbase, ROWS_PER_TILE)],
                    px_h.at[c, pl.ds(rbase, ROWS_PER_TILE)])
    pltpu.sync_copy(aggat.at[pl.ds(rbase, ROWS_PER_TILE)],
                    pat_h.at[c, pl.ds(rbase, ROWS_PER_TILE)])

  return agg_kernel(row, col, x, edge_at)


BLK = 1000


def _tc_combine(px, pat, x, wmx, wmat, wr, bm, br):
  def body(px_r, pat_r, x_r, wmx_r, wmat_r, wr_r, bm_r, br_r, o_r):
    aggx = px_r[0] + px_r[1]
    aggat = pat_r[0] + pat_r[1]
    acc = jnp.dot(aggx, wmx_r[...], preferred_element_type=jnp.float32)
    acc = acc + jnp.dot(aggat, wmat_r[...], preferred_element_type=jnp.float32)
    acc = 0.5 * (acc + bm_r[...])
    acc = acc + jnp.dot(x_r[...], wr_r[...], preferred_element_type=jnp.float32)
    o_r[...] = acc + br_r[...]

  return pl.pallas_call(
      body,
      grid=(N // BLK,),
      in_specs=[
          pl.BlockSpec((NC, BLK, DF), lambda i: (0, i, 0)),
          pl.BlockSpec((NC, BLK, DE + DT), lambda i: (0, i, 0)),
          pl.BlockSpec((BLK, DF), lambda i: (i, 0)),
          pl.BlockSpec((DF, DO), lambda i: (0, 0)),
          pl.BlockSpec((DE + DT, DO), lambda i: (0, 0)),
          pl.BlockSpec((DF, DO), lambda i: (0, 0)),
          pl.BlockSpec((1, DO), lambda i: (0, 0)),
          pl.BlockSpec((1, DO), lambda i: (0, 0)),
      ],
      out_specs=pl.BlockSpec((BLK, DO), lambda i: (i, 0)),
      out_shape=jax.ShapeDtypeStruct((N, DO), jnp.float32),
  )(px, pat, x, wmx, wmat, wr, bm, br)


def kernel(x, edge_index, edge_attr, edge_t, W_m, b_m, W_r, b_r):
  ei = edge_index.astype(jnp.int32)
  row = ei[0]
  col = ei[1]
  edge_at = jnp.concatenate([edge_attr, edge_t], axis=1)
  px, pat = _sc_aggregate(row, col, x, edge_at)
  px = px[:, :N]
  pat = pat[:, :N]
  wmx = W_m[:DF]
  wmat = W_m[DF:]
  bm = b_m.reshape(1, DO)
  br = b_r.reshape(1, DO)
  return _tc_combine(px, pat, x, wmx, wmat, W_r, bm, br)
